# Initial kernel scaffold; baseline (speedup 1.0000x reference)
import jax
import jax.numpy as jnp
from jax.experimental import pallas as pl


def kernel(x, edge_index, W_in, b_in, W_g0, b_g0, ln_g0, ln_b0, W_g1, b_g1, ln_g1, ln_b1, W_ih, W_hh, b_ih, b_hh, W_fc1, b_fc1, W_fc2, b_fc2):
    raise NotImplementedError("write your pallas kernel here")


# speedup vs baseline: 12.7544x; 12.7544x over previous
import functools

import jax
import jax.numpy as jnp
from jax import lax
from jax.experimental import pallas as pl
from jax.experimental.pallas import tpu as pltpu
from jax.experimental.pallas import tpu_sc as plsc

N_NODES = 10000
NP = 10240
F_IN = 128
H = 64
EDGES = 320000
CHUNK = 128
CPT = 80
EP = 32 * CPT * CHUNK
DW = 16
BLK = 512
GRID = NP // BLK
SEQ_L = 10

_sc_mesh = dict(core_axis_name="c", subcore_axis_name="s",
                num_cores=2, num_subcores=16)


def _fill_loop(ref, rows, width, value):
    def body(i, _):
        for k in range(width // 16):
            ref[i, pl.ds(k * 16, 16)] = jnp.full((16,), value, jnp.float32)
        return 0
    lax.fori_loop(0, rows, body, 0)


def _sc_degree_body(dst_hbm, out_hbm, idx_v, ones_v, zbuf_v, acc_sh):
    c = lax.axis_index("c")
    s = lax.axis_index("s")
    wid = c * 16 + s
    _fill_loop(ones_v, CHUNK, DW, 1.0)
    _fill_loop(zbuf_v, 64, DW, 0.0)
    rows_per_sub = NP // 16
    for k in range(rows_per_sub // 64):
        pltpu.sync_copy(zbuf_v, acc_sh.at[pl.ds(s * rows_per_sub + k * 64, 64)])
    plsc.subcore_barrier()
    pltpu.sync_copy(dst_hbm.at[pl.ds(wid * CPT, CPT), :], idx_v)

    def body(j, _):
        pltpu.sync_copy(ones_v, acc_sh.at[idx_v.at[j]], add=True)
        return 0
    lax.fori_loop(0, CPT, body, 0)
    plsc.subcore_barrier()
    pltpu.sync_copy(acc_sh.at[pl.ds(s * rows_per_sub, rows_per_sub)],
                    out_hbm.at[c, pl.ds(s * rows_per_sub, rows_per_sub), :])


def _sc_edge_sum_body(table_hbm, src_hbm, dst_hbm, out_hbm,
                      src_v, dst_v, rows_v, zbuf_v, acc_sh):
    c = lax.axis_index("c")
    s = lax.axis_index("s")
    wid = c * 16 + s
    _fill_loop(zbuf_v, 64, H, 0.0)
    rows_per_sub = NP // 16
    for k in range(rows_per_sub // 64):
        pltpu.sync_copy(zbuf_v, acc_sh.at[pl.ds(s * rows_per_sub + k * 64, 64)])
    plsc.subcore_barrier()
    pltpu.sync_copy(src_hbm.at[pl.ds(wid * CPT, CPT), :], src_v)
    pltpu.sync_copy(dst_hbm.at[pl.ds(wid * CPT, CPT), :], dst_v)

    def body(j, _):
        pltpu.sync_copy(table_hbm.at[src_v.at[j]], rows_v)
        pltpu.sync_copy(rows_v, acc_sh.at[dst_v.at[j]], add=True)
        return 0
    lax.fori_loop(0, CPT, body, 0)
    plsc.subcore_barrier()
    pltpu.sync_copy(acc_sh.at[pl.ds(s * rows_per_sub, rows_per_sub)],
                    out_hbm.at[c, pl.ds(s * rows_per_sub, rows_per_sub), :])


@functools.cache
def _sc_kernels():
    mesh = plsc.VectorSubcoreMesh(**_sc_mesh)
    params = pltpu.CompilerParams(use_tc_tiling_on_sc=False)
    deg = pl.kernel(
        _sc_degree_body,
        out_type=jax.ShapeDtypeStruct((2, NP, DW), jnp.float32),
        mesh=mesh,
        compiler_params=params,
        scratch_types=[
            pltpu.VMEM((CPT, CHUNK), jnp.int32),
            pltpu.VMEM((CHUNK, DW), jnp.float32),
            pltpu.VMEM((64, DW), jnp.float32),
            pltpu.VMEM_SHARED((NP, DW), jnp.float32),
        ],
    )
    edge = pl.kernel(
        _sc_edge_sum_body,
        out_type=jax.ShapeDtypeStruct((2, NP, H), jnp.float32),
        mesh=mesh,
        compiler_params=params,
        scratch_types=[
            pltpu.VMEM((CPT, CHUNK), jnp.int32),
            pltpu.VMEM((CPT, CHUNK), jnp.int32),
            pltpu.VMEM((CHUNK, H), jnp.float32),
            pltpu.VMEM((64, H), jnp.float32),
            pltpu.VMEM_SHARED((NP, H), jnp.float32),
        ],
    )
    return deg, edge


def _dinv_from(dp0, dp1):
    deg = dp0 + dp1 + 1.0
    return lax.rsqrt(jnp.maximum(deg, 1e-12))


def _tc_input_body(x_ref, dp_ref, win_ref, bin_ref, wg0_ref, h0_ref, t0_ref):
    dinv = _dinv_from(dp_ref[0, :, 0], dp_ref[1, :, 0])
    h0 = jnp.maximum(x_ref[...] @ win_ref[...] + bin_ref[...], 0.0)
    h0_ref[...] = h0
    t0_ref[...] = dinv[:, None] * (h0 @ wg0_ref[...])


def _layer_norm_block(g, gamma, beta):
    m = jnp.mean(g, axis=-1, keepdims=True)
    v = jnp.mean((g - m) * (g - m), axis=-1, keepdims=True)
    return (g - m) * lax.rsqrt(v + 1e-5) * gamma + beta


def _tc_gcn_epilogue_body(acc_ref, t_ref, h_ref, dp_ref, bg_ref, lng_ref,
                          lnb_ref, wg_ref, hn_ref, tn_ref):
    dinv = _dinv_from(dp_ref[0, :, 0], dp_ref[1, :, 0])
    ssum = acc_ref[0] + acc_ref[1] + t_ref[...]
    g = dinv[:, None] * ssum + bg_ref[...]
    g = _layer_norm_block(g, lng_ref[...], lnb_ref[...])
    hn = jnp.maximum(g, 0.0) + h_ref[...]
    hn_ref[...] = hn
    tn_ref[...] = dinv[:, None] * (hn @ wg_ref[...])


def _tc_gcn_final_body(acc_ref, t_ref, h_ref, dp_ref, bg_ref, lng_ref,
                       lnb_ref, hn_ref):
    dinv = _dinv_from(dp_ref[0, :, 0], dp_ref[1, :, 0])
    ssum = acc_ref[0] + acc_ref[1] + t_ref[...]
    g = dinv[:, None] * ssum + bg_ref[...]
    g = _layer_norm_block(g, lng_ref[...], lnb_ref[...])
    hn_ref[...] = jnp.maximum(g, 0.0) + h_ref[...]


def _tc_lstm_body(hc_ref, hp_ref, wih_ref, whh_ref, bih_ref, bhh_ref,
                  wf1_ref, bf1_ref, wf2_ref, bf2_ref, out_ref):
    i = pl.program_id(0)
    prev_scale = jnp.where(i > 0, 1.0, 0.0)
    wih = wih_ref[...]
    pc = hc_ref[...] @ wih
    pp = (hp_ref[...] * prev_scale) @ wih
    whh = whh_ref[...]
    bias = bih_ref[...] + bhh_ref[...]
    h = jnp.zeros((BLK, H), jnp.float32)
    c = jnp.zeros((BLK, H), jnp.float32)
    for t in range(SEQ_L):
        sft = SEQ_L - 1 - t
        if sft == 0:
            xp = pc
        else:
            xp = jnp.concatenate([pp[BLK - sft:, :], pc[:BLK - sft, :]], axis=0)
        z = xp + h @ whh + bias
        zi = jax.nn.sigmoid(z[:, :H])
        zf = jax.nn.sigmoid(z[:, H:2 * H])
        zg = jnp.tanh(z[:, 2 * H:3 * H])
        zo = jax.nn.sigmoid(z[:, 3 * H:])
        c = zf * c + zi * zg
        h = zo * jnp.tanh(c)
    r = jnp.maximum(h @ wf1_ref[...] + bf1_ref[...], 0.0)
    out_ref[...] = (r @ wf2_ref[...]) + bf2_ref[...]


def _row_spec(shape):
    return pl.BlockSpec(shape, lambda i: (0, 0))


def kernel(x, edge_index, W_in, b_in, W_g0, b_g0, ln_g0, ln_b0, W_g1, b_g1,
           ln_g1, ln_b1, W_ih, W_hh, b_ih, b_hh, W_fc1, b_fc1, W_fc2, b_fc2):
    f32 = jnp.float32
    n = x.shape[0]
    x_p = jnp.zeros((NP, F_IN), f32).at[:n].set(x)
    src = edge_index[0]
    dst = edge_index[1]
    e = src.shape[0]
    pad = EP - e
    src_p = jnp.concatenate([src, jnp.zeros((pad,), jnp.int32)]).reshape(EP // CHUNK, CHUNK)
    dst_p = jnp.concatenate([dst, jnp.full((pad,), n, jnp.int32)]).reshape(EP // CHUNK, CHUNK)

    sc_degree, sc_edge_sum = _sc_kernels()
    dp = sc_degree(dst_p)

    blk_h = pl.BlockSpec((BLK, H), lambda i: (i, 0))
    blk_dp = pl.BlockSpec((2, BLK, DW), lambda i: (0, i, 0))
    blk_acc = pl.BlockSpec((2, BLK, H), lambda i: (0, i, 0))

    h0, t0 = pl.pallas_call(
        _tc_input_body,
        grid=(GRID,),
        in_specs=[
            pl.BlockSpec((BLK, F_IN), lambda i: (i, 0)),
            blk_dp,
            _row_spec((F_IN, H)),
            _row_spec((1, H)),
            _row_spec((H, H)),
        ],
        out_specs=[blk_h, blk_h],
        out_shape=[jax.ShapeDtypeStruct((NP, H), f32)] * 2,
    )(x_p, dp, W_in, b_in.reshape(1, H), W_g0)

    acc0 = sc_edge_sum(t0, src_p, dst_p)

    h1, t1 = pl.pallas_call(
        _tc_gcn_epilogue_body,
        grid=(GRID,),
        in_specs=[
            blk_acc, blk_h, blk_h, blk_dp,
            _row_spec((1, H)), _row_spec((1, H)), _row_spec((1, H)),
            _row_spec((H, H)),
        ],
        out_specs=[blk_h, blk_h],
        out_shape=[jax.ShapeDtypeStruct((NP, H), f32)] * 2,
    )(acc0, t0, h0, dp, b_g0.reshape(1, H), ln_g0.reshape(1, H),
      ln_b0.reshape(1, H), W_g1)

    acc1 = sc_edge_sum(t1, src_p, dst_p)

    h2 = pl.pallas_call(
        _tc_gcn_final_body,
        grid=(GRID,),
        in_specs=[
            blk_acc, blk_h, blk_h, blk_dp,
            _row_spec((1, H)), _row_spec((1, H)), _row_spec((1, H)),
        ],
        out_specs=blk_h,
        out_shape=jax.ShapeDtypeStruct((NP, H), f32),
    )(acc1, t1, h1, dp, b_g1.reshape(1, H), ln_g1.reshape(1, H),
      ln_b1.reshape(1, H))

    out = pl.pallas_call(
        _tc_lstm_body,
        grid=(GRID,),
        in_specs=[
            blk_h,
            pl.BlockSpec((BLK, H), lambda i: (jnp.maximum(i - 1, 0), 0)),
            _row_spec((H, 4 * H)),
            _row_spec((H, 4 * H)),
            _row_spec((1, 4 * H)),
            _row_spec((1, 4 * H)),
            _row_spec((H, H // 2)),
            _row_spec((1, H // 2)),
            _row_spec((H // 2, 1)),
            _row_spec((1, 1)),
        ],
        out_specs=pl.BlockSpec((BLK, 1), lambda i: (i, 0)),
        out_shape=jax.ShapeDtypeStruct((NP, 1), f32),
    )(h2, h2, W_ih, W_hh, b_ih.reshape(1, 4 * H), b_hh.reshape(1, 4 * H),
      W_fc1, b_fc1.reshape(1, H // 2), W_fc2, b_fc2.reshape(1, 1))

    return out[:n, 0]


# speedup vs baseline: 14.4882x; 1.1359x over previous
import functools

import jax
import jax.numpy as jnp
from jax import lax
from jax.experimental import pallas as pl
from jax.experimental.pallas import tpu as pltpu
from jax.experimental.pallas import tpu_sc as plsc

N_NODES = 10000
NP = 10240
F_IN = 128
H = 64
EDGES = 320000
CHUNK = 128
CPT = 80
EP = 32 * CPT * CHUNK
DW = 16
BLK = 512
GRID = NP // BLK
SEQ_L = 10

_sc_mesh = dict(core_axis_name="c", subcore_axis_name="s",
                num_cores=2, num_subcores=16)


def _fill_loop(ref, rows, width, value):
    def body(i, _):
        for k in range(width // 16):
            ref[i, pl.ds(k * 16, 16)] = jnp.full((16,), value, jnp.float32)
        return 0
    lax.fori_loop(0, rows, body, 0)


def _sc_degree_body(dst_hbm, out_hbm, idx_v, ones_v, zbuf_v, acc_sh):
    c = lax.axis_index("c")
    s = lax.axis_index("s")
    wid = c * 16 + s
    _fill_loop(ones_v, CHUNK, DW, 1.0)
    _fill_loop(zbuf_v, 64, DW, 0.0)
    rows_per_sub = NP // 16
    for k in range(rows_per_sub // 64):
        pltpu.sync_copy(zbuf_v, acc_sh.at[pl.ds(s * rows_per_sub + k * 64, 64)])
    plsc.subcore_barrier()
    pltpu.sync_copy(dst_hbm.at[pl.ds(wid * CPT, CPT), :], idx_v)

    def body(j, _):
        pltpu.sync_copy(ones_v, acc_sh.at[idx_v.at[j]], add=True)
        return 0
    lax.fori_loop(0, CPT, body, 0)
    plsc.subcore_barrier()
    pltpu.sync_copy(acc_sh.at[pl.ds(s * rows_per_sub, rows_per_sub)],
                    out_hbm.at[c, pl.ds(s * rows_per_sub, rows_per_sub), :])


NB = 8
NG = CPT // NB


def _sc_edge_sum_body(table_hbm, src_hbm, dst_hbm, out_hbm,
                      src_v, dst_v, rows_v, zbuf_v, acc_sh, gsem, ssem):
    c = lax.axis_index("c")
    s = lax.axis_index("s")
    wid = c * 16 + s
    _fill_loop(zbuf_v, 64, H, 0.0)
    rows_per_sub = NP // 16
    for k in range(rows_per_sub // 64):
        pltpu.sync_copy(zbuf_v, acc_sh.at[pl.ds(s * rows_per_sub + k * 64, 64)])
    plsc.subcore_barrier()
    pltpu.sync_copy(src_hbm.at[pl.ds(wid * CPT, CPT), :], src_v)
    pltpu.sync_copy(dst_hbm.at[pl.ds(wid * CPT, CPT), :], dst_v)

    def gather(j, b):
        pltpu.async_copy(table_hbm.at[src_v.at[j]], rows_v.at[b], gsem)

    def gather_wait(j, b):
        pltpu.make_async_copy(table_hbm.at[src_v.at[j]], rows_v.at[b],
                              gsem).wait()

    def scatter(j, b):
        pltpu.async_copy(rows_v.at[b], acc_sh.at[dst_v.at[j]], ssem, add=True)

    def scatter_wait(j, b):
        pltpu.make_async_copy(rows_v.at[b], acc_sh.at[dst_v.at[j]],
                              ssem).wait()

    for b in range(NB):
        gather(b, b)

    def group(g, _):
        for b in range(NB):
            j = g * NB + b
            gather_wait(j, b)
            scatter(j, b)
        for b in range(NB):
            j = g * NB + b
            scatter_wait(j, b)
            gather(j + NB, b)
        return 0
    lax.fori_loop(0, NG - 1, group, 0)
    g = NG - 1
    for b in range(NB):
        j = g * NB + b
        gather_wait(j, b)
        scatter(j, b)
    for b in range(NB):
        j = g * NB + b
        scatter_wait(j, b)
    plsc.subcore_barrier()
    pltpu.sync_copy(acc_sh.at[pl.ds(s * rows_per_sub, rows_per_sub)],
                    out_hbm.at[c, pl.ds(s * rows_per_sub, rows_per_sub), :])


@functools.cache
def _sc_kernels():
    mesh = plsc.VectorSubcoreMesh(**_sc_mesh)
    params = pltpu.CompilerParams(use_tc_tiling_on_sc=False)
    deg = pl.kernel(
        _sc_degree_body,
        out_type=jax.ShapeDtypeStruct((2, NP, DW), jnp.float32),
        mesh=mesh,
        compiler_params=params,
        scratch_types=[
            pltpu.VMEM((CPT, CHUNK), jnp.int32),
            pltpu.VMEM((CHUNK, DW), jnp.float32),
            pltpu.VMEM((64, DW), jnp.float32),
            pltpu.VMEM_SHARED((NP, DW), jnp.float32),
        ],
    )
    edge = pl.kernel(
        _sc_edge_sum_body,
        out_type=jax.ShapeDtypeStruct((2, NP, H), jnp.float32),
        mesh=mesh,
        compiler_params=params,
        scratch_types=[
            pltpu.VMEM((CPT, CHUNK), jnp.int32),
            pltpu.VMEM((CPT, CHUNK), jnp.int32),
            pltpu.VMEM((NB, CHUNK, H), jnp.float32),
            pltpu.VMEM((64, H), jnp.float32),
            pltpu.VMEM_SHARED((NP, H), jnp.float32),
            pltpu.SemaphoreType.DMA,
            pltpu.SemaphoreType.DMA,
        ],
    )
    return deg, edge


def _dinv_from(dp0, dp1):
    deg = dp0 + dp1 + 1.0
    return lax.rsqrt(jnp.maximum(deg, 1e-12))


def _tc_input_body(x_ref, dp_ref, win_ref, bin_ref, wg0_ref, h0_ref, t0_ref):
    dinv = _dinv_from(dp_ref[0, :, 0], dp_ref[1, :, 0])
    h0 = jnp.maximum(x_ref[...] @ win_ref[...] + bin_ref[...], 0.0)
    h0_ref[...] = h0
    t0_ref[...] = dinv[:, None] * (h0 @ wg0_ref[...])


def _layer_norm_block(g, gamma, beta):
    m = jnp.mean(g, axis=-1, keepdims=True)
    v = jnp.mean((g - m) * (g - m), axis=-1, keepdims=True)
    return (g - m) * lax.rsqrt(v + 1e-5) * gamma + beta


def _tc_gcn_epilogue_body(acc_ref, t_ref, h_ref, dp_ref, bg_ref, lng_ref,
                          lnb_ref, wg_ref, hn_ref, tn_ref):
    dinv = _dinv_from(dp_ref[0, :, 0], dp_ref[1, :, 0])
    ssum = acc_ref[0] + acc_ref[1] + t_ref[...]
    g = dinv[:, None] * ssum + bg_ref[...]
    g = _layer_norm_block(g, lng_ref[...], lnb_ref[...])
    hn = jnp.maximum(g, 0.0) + h_ref[...]
    hn_ref[...] = hn
    tn_ref[...] = dinv[:, None] * (hn @ wg_ref[...])


def _tc_gcn_final_body(acc_ref, t_ref, h_ref, dp_ref, bg_ref, lng_ref,
                       lnb_ref, hn_ref):
    dinv = _dinv_from(dp_ref[0, :, 0], dp_ref[1, :, 0])
    ssum = acc_ref[0] + acc_ref[1] + t_ref[...]
    g = dinv[:, None] * ssum + bg_ref[...]
    g = _layer_norm_block(g, lng_ref[...], lnb_ref[...])
    hn_ref[...] = jnp.maximum(g, 0.0) + h_ref[...]


def _tc_lstm_body(hc_ref, hp_ref, wih_ref, whh_ref, bih_ref, bhh_ref,
                  wf1_ref, bf1_ref, wf2_ref, bf2_ref, out_ref):
    i = pl.program_id(0)
    prev_scale = jnp.where(i > 0, 1.0, 0.0)
    wih = wih_ref[...]
    pc = hc_ref[...] @ wih
    pp = (hp_ref[...] * prev_scale) @ wih
    whh = whh_ref[...]
    bias = bih_ref[...] + bhh_ref[...]
    h = jnp.zeros((BLK, H), jnp.float32)
    c = jnp.zeros((BLK, H), jnp.float32)
    for t in range(SEQ_L):
        sft = SEQ_L - 1 - t
        if sft == 0:
            xp = pc
        else:
            xp = jnp.concatenate([pp[BLK - sft:, :], pc[:BLK - sft, :]], axis=0)
        z = xp + h @ whh + bias
        zi = jax.nn.sigmoid(z[:, :H])
        zf = jax.nn.sigmoid(z[:, H:2 * H])
        zg = jnp.tanh(z[:, 2 * H:3 * H])
        zo = jax.nn.sigmoid(z[:, 3 * H:])
        c = zf * c + zi * zg
        h = zo * jnp.tanh(c)
    r = jnp.maximum(h @ wf1_ref[...] + bf1_ref[...], 0.0)
    out_ref[...] = (r @ wf2_ref[...]) + bf2_ref[...]


def _row_spec(shape):
    return pl.BlockSpec(shape, lambda i: (0, 0))


def kernel(x, edge_index, W_in, b_in, W_g0, b_g0, ln_g0, ln_b0, W_g1, b_g1,
           ln_g1, ln_b1, W_ih, W_hh, b_ih, b_hh, W_fc1, b_fc1, W_fc2, b_fc2):
    f32 = jnp.float32
    n = x.shape[0]
    x_p = jnp.zeros((NP, F_IN), f32).at[:n].set(x)
    src = edge_index[0]
    dst = edge_index[1]
    e = src.shape[0]
    pad = EP - e
    src_p = jnp.concatenate([src, jnp.zeros((pad,), jnp.int32)]).reshape(EP // CHUNK, CHUNK)
    dst_p = jnp.concatenate([dst, jnp.full((pad,), n, jnp.int32)]).reshape(EP // CHUNK, CHUNK)

    sc_degree, sc_edge_sum = _sc_kernels()
    dp = sc_degree(dst_p)

    blk_h = pl.BlockSpec((BLK, H), lambda i: (i, 0))
    blk_dp = pl.BlockSpec((2, BLK, DW), lambda i: (0, i, 0))
    blk_acc = pl.BlockSpec((2, BLK, H), lambda i: (0, i, 0))

    h0, t0 = pl.pallas_call(
        _tc_input_body,
        grid=(GRID,),
        in_specs=[
            pl.BlockSpec((BLK, F_IN), lambda i: (i, 0)),
            blk_dp,
            _row_spec((F_IN, H)),
            _row_spec((1, H)),
            _row_spec((H, H)),
        ],
        out_specs=[blk_h, blk_h],
        out_shape=[jax.ShapeDtypeStruct((NP, H), f32)] * 2,
    )(x_p, dp, W_in, b_in.reshape(1, H), W_g0)

    acc0 = sc_edge_sum(t0, src_p, dst_p)

    h1, t1 = pl.pallas_call(
        _tc_gcn_epilogue_body,
        grid=(GRID,),
        in_specs=[
            blk_acc, blk_h, blk_h, blk_dp,
            _row_spec((1, H)), _row_spec((1, H)), _row_spec((1, H)),
            _row_spec((H, H)),
        ],
        out_specs=[blk_h, blk_h],
        out_shape=[jax.ShapeDtypeStruct((NP, H), f32)] * 2,
    )(acc0, t0, h0, dp, b_g0.reshape(1, H), ln_g0.reshape(1, H),
      ln_b0.reshape(1, H), W_g1)

    acc1 = sc_edge_sum(t1, src_p, dst_p)

    h2 = pl.pallas_call(
        _tc_gcn_final_body,
        grid=(GRID,),
        in_specs=[
            blk_acc, blk_h, blk_h, blk_dp,
            _row_spec((1, H)), _row_spec((1, H)), _row_spec((1, H)),
        ],
        out_specs=blk_h,
        out_shape=jax.ShapeDtypeStruct((NP, H), f32),
    )(acc1, t1, h1, dp, b_g1.reshape(1, H), ln_g1.reshape(1, H),
      ln_b1.reshape(1, H))

    out = pl.pallas_call(
        _tc_lstm_body,
        grid=(GRID,),
        in_specs=[
            blk_h,
            pl.BlockSpec((BLK, H), lambda i: (jnp.maximum(i - 1, 0), 0)),
            _row_spec((H, 4 * H)),
            _row_spec((H, 4 * H)),
            _row_spec((1, 4 * H)),
            _row_spec((1, 4 * H)),
            _row_spec((H, H // 2)),
            _row_spec((1, H // 2)),
            _row_spec((H // 2, 1)),
            _row_spec((1, 1)),
        ],
        out_specs=pl.BlockSpec((BLK, 1), lambda i: (i, 0)),
        out_shape=jax.ShapeDtypeStruct((NP, 1), f32),
    )(h2, h2, W_ih, W_hh, b_ih.reshape(1, 4 * H), b_hh.reshape(1, 4 * H),
      W_fc1, b_fc1.reshape(1, H // 2), W_fc2, b_fc2.reshape(1, 1))

    return out[:n, 0]
